# baseline (speedup 1.0000x reference)
import functools

import jax
import jax.numpy as jnp
from jax import lax
from jax.experimental import pallas as pl
from jax.experimental.pallas import tpu as pltpu
from jax.experimental.pallas import tpu_sc as plsc

CHUNK = 40
ROW_UNROLL = 1
JGROUP = 16
NBUF = 3


def kernel(features, w0, w1):
    n, d = features.shape
    info = plsc.get_sparse_core_info()
    nc, ns = info.num_cores, info.num_subcores
    nw = nc * ns
    num_chunks = n // CHUNK
    nvec = d // 16
    per = num_chunks // nw
    rem = num_chunks % nw
    kmax = per + 2
    kmax = ((kmax + NBUF - 1) // NBUF) * NBUF

    mesh = plsc.VectorSubcoreMesh(core_axis_name="c", subcore_axis_name="s")

    @functools.partial(
        pl.kernel,
        mesh=mesh,
        out_type=jax.ShapeDtypeStruct((n, d), jnp.float32),
        scratch_types=(
            [pltpu.VMEM((CHUNK, d), jnp.float32) for _ in range(NBUF)]
            + [pltpu.VMEM((d,), jnp.float32) for _ in range(2)]
            + [pltpu.SemaphoreType.DMA for _ in range(2 * NBUF)]
        ),
    )
    def sc_fwd(feat_hbm, w0_hbm, w1_hbm, out_hbm, b0, b1, b2, w0v, w1v,
               si0, si1, si2, so0, so1, so2):
        bufs = (b0, b1, b2)
        sin = (si0, si1, si2)
        sout = (so0, so1, so2)
        wid = lax.axis_index("s") * nc + lax.axis_index("c")

        def start_in(k, b):
            c = jnp.where(k < per, wid * per + k, nw * per + wid)
            c = jnp.where(k < 0, -1, c)

            @pl.when(jnp.logical_and(c >= 0, jnp.where(k < per, True, jnp.logical_and(k == per, wid < rem))))
            def _():
                pltpu.async_copy(
                    feat_hbm.at[pl.ds(c * CHUNK, CHUNK), :], bufs[b], sin[b]
                )

        def wait_in(k, b):
            c = jnp.where(k < per, wid * per + k, nw * per + wid)
            c = jnp.where(k < 0, -1, c)

            @pl.when(jnp.logical_and(c >= 0, jnp.where(k < per, True, jnp.logical_and(k == per, wid < rem))))
            def _():
                pltpu.make_async_copy(
                    feat_hbm.at[pl.ds(c * CHUNK, CHUNK), :], bufs[b], sin[b]
                ).wait()

        def start_out(k, b):
            c = jnp.where(k < per, wid * per + k, nw * per + wid)
            c = jnp.where(k < 0, -1, c)

            @pl.when(jnp.logical_and(c >= 0, jnp.where(k < per, True, jnp.logical_and(k == per, wid < rem))))
            def _():
                pltpu.async_copy(
                    bufs[b], out_hbm.at[pl.ds(c * CHUNK, CHUNK), :], sout[b]
                )

        def wait_out(k, b):
            c = jnp.where(k < per, wid * per + k, nw * per + wid)
            c = jnp.where(k < 0, -1, c)

            @pl.when(jnp.logical_and(c >= 0, jnp.where(k < per, True, jnp.logical_and(k == per, wid < rem))))
            def _():
                pltpu.make_async_copy(
                    bufs[b], out_hbm.at[pl.ds(c * CHUNK, CHUNK), :], sout[b]
                ).wait()

        def compute(k, b):
            c = jnp.where(k < per, wid * per + k, nw * per + wid)
            c = jnp.where(k < 0, -1, c)

            @pl.when(jnp.logical_and(c >= 0, jnp.where(k < per, True, jnp.logical_and(k == per, wid < rem))))
            def _():
                buf = bufs[b]
                for jg in range(nvec // JGROUP):
                    w0s = [w0v[pl.ds((jg * JGROUP + t) * 16, 16)]
                           for t in range(JGROUP)]
                    w1s = [w1v[pl.ds((jg * JGROUP + t) * 16, 16)]
                           for t in range(JGROUP)]

                    @plsc.parallel_loop(0, CHUNK, 1, unroll=ROW_UNROLL)
                    def _rows(r, jg=jg, w0s=w0s, w1s=w1s, buf=buf):
                        for t in range(JGROUP):
                            j = jg * JGROUP + t
                            x = buf[r, pl.ds(j * 16, 16)]
                            buf[r, pl.ds(j * 16, 16)] = (
                                jnp.maximum(x * w0s[t], 0.0) * w1s[t]
                            )

        start_in(0, 0)
        start_in(1, 1)
        pltpu.sync_copy(w0_hbm, w0v)
        pltpu.sync_copy(w1_hbm, w1v)

        def step(k0, _):
            for b in range(NBUF):
                k = k0 * NBUF + b
                wait_in(k, b)
                wait_out(k - 1, (b - 1) % NBUF)
                start_in(k + 2, (b + 2) % NBUF)
                compute(k, b)
                start_out(k, b)
            return _

        lax.fori_loop(0, kmax // NBUF, step, None)

    return sc_fwd(features, w0, w1)


# speedup vs baseline: 1.0121x; 1.0121x over previous
import functools

import jax
import jax.numpy as jnp
from jax import lax
from jax.experimental import pallas as pl
from jax.experimental.pallas import tpu as pltpu
from jax.experimental.pallas import tpu_sc as plsc

CHUNK = 40
ROW_UNROLL = 1
JGROUP = 16
NBUF = 3


def kernel(features, w0, w1):
    n, d = features.shape
    info = plsc.get_sparse_core_info()
    nc, ns = info.num_cores, info.num_subcores
    nw = nc * ns
    num_chunks = n // CHUNK
    nvec = d // 16
    per = num_chunks // nw
    rem = num_chunks % nw
    kmax = per + 2
    kmax = ((kmax + NBUF - 1) // NBUF) * NBUF

    mesh = plsc.VectorSubcoreMesh(core_axis_name="c", subcore_axis_name="s")

    @functools.partial(
        pl.kernel,
        mesh=mesh,
        out_type=jax.ShapeDtypeStruct((n, d), jnp.float32),
        scratch_types=(
            [pltpu.VMEM((CHUNK, d), jnp.float32) for _ in range(NBUF)]
            + [pltpu.VMEM((d,), jnp.float32) for _ in range(2)]
            + [pltpu.SemaphoreType.DMA for _ in range(2 * NBUF)]
        ),
    )
    def sc_fwd(feat_hbm, w0_hbm, w1_hbm, out_hbm, b0, b1, b2, w0v, w1v,
               si0, si1, si2, so0, so1, so2):
        bufs = (b0, b1, b2)
        sin = (si0, si1, si2)
        sout = (so0, so1, so2)
        wid = lax.axis_index("s") * nc + lax.axis_index("c")

        def start_in(k, b):
            c = jnp.where(k < per, wid * per + k, nw * per + wid)
            c = jnp.where(k < 0, -1, c)

            @pl.when(jnp.logical_and(c >= 0, jnp.where(k < per, True, jnp.logical_and(k == per, wid < rem))))
            def _():
                pltpu.async_copy(
                    feat_hbm.at[pl.ds(c * CHUNK, CHUNK), :], bufs[b], sin[b]
                )

        def wait_in(k, b):
            c = jnp.where(k < per, wid * per + k, nw * per + wid)
            c = jnp.where(k < 0, -1, c)

            @pl.when(jnp.logical_and(c >= 0, jnp.where(k < per, True, jnp.logical_and(k == per, wid < rem))))
            def _():
                pltpu.make_async_copy(
                    feat_hbm.at[pl.ds(c * CHUNK, CHUNK), :], bufs[b], sin[b]
                ).wait()

        def start_out(k, b):
            c = jnp.where(k < per, wid * per + k, nw * per + wid)
            c = jnp.where(k < 0, -1, c)

            @pl.when(jnp.logical_and(c >= 0, jnp.where(k < per, True, jnp.logical_and(k == per, wid < rem))))
            def _():
                pltpu.async_copy(
                    bufs[b], out_hbm.at[pl.ds(c * CHUNK, CHUNK), :], sout[b]
                )

        def wait_out(k, b):
            c = jnp.where(k < per, wid * per + k, nw * per + wid)
            c = jnp.where(k < 0, -1, c)

            @pl.when(jnp.logical_and(c >= 0, jnp.where(k < per, True, jnp.logical_and(k == per, wid < rem))))
            def _():
                pltpu.make_async_copy(
                    bufs[b], out_hbm.at[pl.ds(c * CHUNK, CHUNK), :], sout[b]
                ).wait()

        def compute(k, b):
            c = jnp.where(k < per, wid * per + k, nw * per + wid)
            c = jnp.where(k < 0, -1, c)

            @pl.when(jnp.logical_and(c >= 0, jnp.where(k < per, True, jnp.logical_and(k == per, wid < rem))))
            def _():
                buf = bufs[b]
                for jg in range(nvec // JGROUP):
                    w0s = [w0v[pl.ds((jg * JGROUP + t) * 16, 16)]
                           for t in range(JGROUP)]
                    w1s = [w1v[pl.ds((jg * JGROUP + t) * 16, 16)]
                           for t in range(JGROUP)]

                    @plsc.parallel_loop(0, CHUNK, 1, unroll=ROW_UNROLL)
                    def _rows(r, jg=jg, w0s=w0s, w1s=w1s, buf=buf):
                        for t in range(JGROUP):
                            j = jg * JGROUP + t
                            x = buf[r, pl.ds(j * 16, 16)]
                            buf[r, pl.ds(j * 16, 16)] = (
                                jnp.maximum(x * w0s[t], 0.0) * w1s[t]
                            )

        start_in(0, 0)
        start_in(1, 1)
        pltpu.sync_copy(w0_hbm, w0v)
        pltpu.sync_copy(w1_hbm, w1v)

        def step(k0, _):
            for b in range(NBUF):
                k = k0 * NBUF + b
                wait_in(k, b)
                compute(k, b)
                start_out(k, b)
                wait_out(k - 1, (b - 1) % NBUF)
                start_in(k + 2, (b + 2) % NBUF)
            return _

        lax.fori_loop(0, kmax // NBUF, step, None)

    return sc_fwd(features, w0, w1)
